# Initial kernel scaffold; baseline (speedup 1.0000x reference)
import jax
import jax.numpy as jnp
from jax.experimental import pallas as pl


def kernel(values, weights, edge_index):
    raise NotImplementedError("write your pallas kernel here")


# speedup vs baseline: 138.8603x; 138.8603x over previous
import functools

import jax
import jax.numpy as jnp
from jax import lax
from jax.experimental import pallas as pl
from jax.experimental.pallas import tpu as pltpu
from jax.experimental.pallas import tpu_sc as plsc

N = 100000
E = 6400000
NP = 100352
ROWS = E // 128
KB = 16
CHUNKS = ROWS // KB
NW = 32
Q, REM = divmod(CHUNKS, NW)
SLICE = NP // 16

_mesh = plsc.VectorSubcoreMesh(core_axis_name="c", subcore_axis_name="s")


@functools.partial(
    pl.kernel,
    mesh=_mesh,
    out_type=jax.ShapeDtypeStruct((2, NP), jnp.float32),
    compiler_params=pltpu.CompilerParams(needs_layout_passes=False),
    scratch_types=[
        pltpu.VMEM((NP,), jnp.float32),
        pltpu.VMEM((SLICE,), jnp.float32),
        pltpu.VMEM((KB, 128), jnp.int32),
        pltpu.VMEM((KB, 128), jnp.int32),
        pltpu.VMEM((KB, 128), jnp.float32),
        pltpu.VMEM((KB, 128), jnp.float32),
        pltpu.VMEM_SHARED((NP,), jnp.float32),
    ],
)
def _sc_scatter(t_hbm, ei_hbm, w_hbm, out_hbm,
                t_local, zbuf, srcb, dstb, wb, msgb, acc):
    cid = lax.axis_index("c")
    sid = lax.axis_index("s")
    wid = sid * 2 + cid

    def _zero(i, carry):
        zbuf[pl.ds(i * 16, 16)] = jnp.zeros((16,), jnp.float32)
        return carry
    lax.fori_loop(0, SLICE // 16, _zero, 0)
    pltpu.sync_copy(zbuf, acc.at[pl.ds(sid * SLICE, SLICE)])

    pltpu.sync_copy(t_hbm, t_local)
    plsc.subcore_barrier()

    nchunks = Q + jnp.where(wid < REM, 1, 0)
    base = wid * Q + jnp.minimum(wid, REM)

    def _chunk(c, carry):
        row0 = (base + c) * KB
        pltpu.sync_copy(ei_hbm.at[0, pl.ds(row0, KB)], srcb)
        pltpu.sync_copy(ei_hbm.at[1, pl.ds(row0, KB)], dstb)
        pltpu.sync_copy(w_hbm.at[pl.ds(row0, KB)], wb)

        def _row(j, rc):
            for l in range(8):
                idx = srcb[j, pl.ds(l * 16, 16)]
                tv = plsc.load_gather(t_local, [idx])
                msgb[j, pl.ds(l * 16, 16)] = tv * wb[j, pl.ds(l * 16, 16)]
            return rc
        lax.fori_loop(0, KB, _row, 0)

        def _scat(j, rc):
            pltpu.sync_copy(msgb.at[j], acc.at[dstb.at[j]], add=True)
            return rc
        lax.fori_loop(0, KB, _scat, 0)
        return carry
    lax.fori_loop(0, nchunks, _chunk, 0)

    plsc.subcore_barrier()
    pltpu.sync_copy(acc.at[pl.ds(sid * SLICE, SLICE)], zbuf)
    pltpu.sync_copy(zbuf, out_hbm.at[cid, pl.ds(sid * SLICE, SLICE)])


def _tanh_body(v_ref, o_ref):
    o_ref[...] = jnp.tanh(v_ref[...])


def _finish_body(v_ref, p_ref, o_ref):
    o_ref[...] = v_ref[...] - p_ref[0] - p_ref[1]


def kernel(values, weights, edge_index):
    v2 = jnp.pad(values, (0, NP - N)).reshape(NP // 128, 128)
    t2 = pl.pallas_call(
        _tanh_body,
        out_shape=jax.ShapeDtypeStruct((NP // 128, 128), jnp.float32),
    )(v2)
    ei3 = edge_index.reshape(2, ROWS, 128)
    w2 = weights.reshape(ROWS, 128)
    preds = _sc_scatter(t2.reshape(NP), ei3, w2)
    e2 = pl.pallas_call(
        _finish_body,
        out_shape=jax.ShapeDtypeStruct((NP // 128, 128), jnp.float32),
    )(v2, preds.reshape(2, NP // 128, 128))
    return e2.reshape(NP)[:N]


# speedup vs baseline: 287.2595x; 2.0687x over previous
import functools

import jax
import jax.numpy as jnp
from jax import lax
from jax.experimental import pallas as pl
from jax.experimental.pallas import tpu as pltpu
from jax.experimental.pallas import tpu_sc as plsc

N = 100000
E = 6400000
NP = 100352
ROWS = E // 128
KB = 16
CHUNKS = ROWS // KB
NW = 32
MAIN = CHUNKS // NW
EXTRA = CHUNKS - NW * MAIN
SLICE = NP // 16

_mesh = plsc.VectorSubcoreMesh(core_axis_name="c", subcore_axis_name="s")


@functools.partial(
    pl.kernel,
    mesh=_mesh,
    out_type=jax.ShapeDtypeStruct((2, NP), jnp.float32),
    compiler_params=pltpu.CompilerParams(needs_layout_passes=False),
    scratch_types=[
        pltpu.VMEM((NP,), jnp.float32),
        pltpu.VMEM((SLICE,), jnp.float32),
        pltpu.VMEM((KB, 128), jnp.int32),
        pltpu.VMEM((KB, 128), jnp.int32),
        pltpu.VMEM((KB, 128), jnp.float32),
        pltpu.VMEM((KB, 128), jnp.float32),
        pltpu.VMEM((KB, 128), jnp.int32),
        pltpu.VMEM((KB, 128), jnp.int32),
        pltpu.VMEM((KB, 128), jnp.float32),
        pltpu.VMEM((KB, 128), jnp.float32),
        pltpu.VMEM_SHARED((NP,), jnp.float32),
        pltpu.SemaphoreType.DMA,
        pltpu.SemaphoreType.DMA,
        pltpu.SemaphoreType.DMA,
    ],
)
def _sc_scatter(t_hbm, ei_hbm, w_hbm, out_hbm,
                t_local, zbuf,
                srcb0, dstb0, wb0, msgb0,
                srcb1, dstb1, wb1, msgb1,
                acc, sem_in0, sem_in1, sem_sc):
    srcb = (srcb0, srcb1)
    dstb = (dstb0, dstb1)
    wb = (wb0, wb1)
    msgb = (msgb0, msgb1)
    sem_in = (sem_in0, sem_in1)

    cid = lax.axis_index("c")
    sid = lax.axis_index("s")
    wid = sid * 2 + cid

    def _zero(i, carry):
        zbuf[pl.ds(i * 16, 16)] = jnp.zeros((16,), jnp.float32)
        return carry
    lax.fori_loop(0, SLICE // 16, _zero, 0)
    pltpu.sync_copy(zbuf, acc.at[pl.ds(sid * SLICE, SLICE)])

    pltpu.sync_copy(t_hbm, t_local)
    plsc.subcore_barrier()

    base = wid * MAIN

    def start_in(b, c):
        row0 = c * KB
        pltpu.async_copy(ei_hbm.at[0, pl.ds(row0, KB)], srcb[b], sem_in[b])
        pltpu.async_copy(ei_hbm.at[1, pl.ds(row0, KB)], dstb[b], sem_in[b])
        pltpu.async_copy(w_hbm.at[pl.ds(row0, KB)], wb[b], sem_in[b])

    def wait_in(b):
        pltpu.make_async_copy(ei_hbm.at[0, pl.ds(0, KB)], srcb[b], sem_in[b]).wait()
        pltpu.make_async_copy(ei_hbm.at[1, pl.ds(0, KB)], dstb[b], sem_in[b]).wait()
        pltpu.make_async_copy(w_hbm.at[pl.ds(0, KB)], wb[b], sem_in[b]).wait()

    def compute(b):
        def _row(j, rc):
            for l in range(8):
                idx = srcb[b][j, pl.ds(l * 16, 16)]
                tv = plsc.load_gather(t_local, [idx])
                msgb[b][j, pl.ds(l * 16, 16)] = tv * wb[b][j, pl.ds(l * 16, 16)]
            return rc
        lax.fori_loop(0, KB, _row, 0)

    def scatter(b):
        return [
            pltpu.async_copy(msgb[b].at[j], acc.at[dstb[b].at[j]], sem_sc,
                             add=True)
            for j in range(KB)
        ]

    def drain(descs):
        for d in descs:
            d.wait()

    start_in(0, base)

    @pl.loop(0, MAIN - 1, step=2)
    def _main(g):
        for b in (0, 1):
            c = base + g + b
            wait_in(b)
            compute(b)
            sc_descs = scatter(b)
            start_in(1 - b, c + 1)
            drain(sc_descs)

    wait_in(0)
    compute(0)
    drain(scatter(0))

    @pl.when(wid < EXTRA)
    def _extra():
        row0 = (NW * MAIN + wid) * KB
        pltpu.sync_copy(ei_hbm.at[0, pl.ds(row0, KB)], srcb0)
        pltpu.sync_copy(ei_hbm.at[1, pl.ds(row0, KB)], dstb0)
        pltpu.sync_copy(w_hbm.at[pl.ds(row0, KB)], wb0)
        compute(0)
        drain(scatter(0))

    plsc.subcore_barrier()
    pltpu.sync_copy(acc.at[pl.ds(sid * SLICE, SLICE)], zbuf)
    pltpu.sync_copy(zbuf, out_hbm.at[cid, pl.ds(sid * SLICE, SLICE)])


def _tanh_body(v_ref, o_ref):
    o_ref[...] = jnp.tanh(v_ref[...])


def _finish_body(v_ref, p_ref, o_ref):
    o_ref[...] = v_ref[...] - p_ref[0] - p_ref[1]


def kernel(values, weights, edge_index):
    v2 = jnp.pad(values, (0, NP - N)).reshape(NP // 128, 128)
    t2 = pl.pallas_call(
        _tanh_body,
        out_shape=jax.ShapeDtypeStruct((NP // 128, 128), jnp.float32),
    )(v2)
    ei3 = edge_index.reshape(2, ROWS, 128)
    w2 = weights.reshape(ROWS, 128)
    preds = _sc_scatter(t2.reshape(NP), ei3, w2)
    e2 = pl.pallas_call(
        _finish_body,
        out_shape=jax.ShapeDtypeStruct((NP // 128, 128), jnp.float32),
    )(v2, preds.reshape(2, NP // 128, 128))
    return e2.reshape(NP)[:N]


# speedup vs baseline: 454.8603x; 1.5834x over previous
import functools

import jax
import jax.numpy as jnp
from jax import lax
from jax.experimental import pallas as pl
from jax.experimental.pallas import tpu as pltpu
from jax.experimental.pallas import tpu_sc as plsc

N = 100000
E = 6400000
NP = 100352
CE = 2048
CHUNKS = E // CE
NW = 32
MAIN = CHUNKS // NW
EXTRA = CHUNKS - NW * MAIN
SLICE = NP // 16

_mesh = plsc.VectorSubcoreMesh(core_axis_name="c", subcore_axis_name="s")


@functools.partial(
    pl.kernel,
    mesh=_mesh,
    out_type=(jax.ShapeDtypeStruct((NP,), jnp.float32),
              jax.ShapeDtypeStruct((NP,), jnp.float32)),
    compiler_params=pltpu.CompilerParams(needs_layout_passes=False),
    scratch_types=[
        pltpu.VMEM((NP,), jnp.float32),
        pltpu.VMEM((SLICE,), jnp.float32),
        pltpu.VMEM((CE,), jnp.int32),
        pltpu.VMEM((CE,), jnp.int32),
        pltpu.VMEM((CE,), jnp.float32),
        pltpu.VMEM((CE,), jnp.float32),
        pltpu.VMEM((CE,), jnp.int32),
        pltpu.VMEM((CE,), jnp.int32),
        pltpu.VMEM((CE,), jnp.float32),
        pltpu.VMEM((CE,), jnp.float32),
        pltpu.VMEM_SHARED((NP,), jnp.float32),
        pltpu.SemaphoreType.DMA,
        pltpu.SemaphoreType.DMA,
        pltpu.SemaphoreType.DMA,
    ],
)
def _sc_scatter(t_hbm, ei_hbm, w_hbm, out0_hbm, out1_hbm,
                t_local, zbuf,
                srcb0, dstb0, wb0, msgb0,
                srcb1, dstb1, wb1, msgb1,
                acc, sem_in0, sem_in1, sem_sc):
    srcb = (srcb0, srcb1)
    dstb = (dstb0, dstb1)
    wb = (wb0, wb1)
    msgb = (msgb0, msgb1)
    sem_in = (sem_in0, sem_in1)

    cid = lax.axis_index("c")
    sid = lax.axis_index("s")
    wid = sid * 2 + cid

    def _zero(i, carry):
        zbuf[pl.ds(i * 16, 16)] = jnp.zeros((16,), jnp.float32)
        return carry
    lax.fori_loop(0, SLICE // 16, _zero, 0)
    pltpu.sync_copy(zbuf, acc.at[pl.ds(sid * SLICE, SLICE)])

    pltpu.sync_copy(t_hbm, t_local)
    plsc.subcore_barrier()

    base = wid * MAIN

    def start_in(b, c):
        e0 = c * CE
        pltpu.async_copy(ei_hbm.at[0, pl.ds(e0, CE)], srcb[b], sem_in[b])
        pltpu.async_copy(ei_hbm.at[1, pl.ds(e0, CE)], dstb[b], sem_in[b])
        pltpu.async_copy(w_hbm.at[pl.ds(e0, CE)], wb[b], sem_in[b])

    def wait_in(b):
        pltpu.make_async_copy(ei_hbm.at[0, pl.ds(0, CE)], srcb[b], sem_in[b]).wait()
        pltpu.make_async_copy(ei_hbm.at[1, pl.ds(0, CE)], dstb[b], sem_in[b]).wait()
        pltpu.make_async_copy(w_hbm.at[pl.ds(0, CE)], wb[b], sem_in[b]).wait()

    def compute(b):
        @plsc.parallel_loop(0, CE, 16, unroll=4)
        def _it(i):
            idx = srcb[b][pl.ds(i, 16)]
            tv = plsc.load_gather(t_local, [idx])
            msgb[b][pl.ds(i, 16)] = tv * wb[b][pl.ds(i, 16)]

    def scatter(b):
        return pltpu.async_copy(msgb[b], acc.at[dstb[b]], sem_sc, add=True)

    start_in(0, base)

    @pl.loop(0, MAIN - 1, step=2)
    def _main(g):
        for b in (0, 1):
            c = base + g + b
            wait_in(b)
            compute(b)
            d = scatter(b)
            start_in(1 - b, c + 1)
            d.wait()

    wait_in(0)
    compute(0)
    scatter(0).wait()

    @pl.when(wid < EXTRA)
    def _extra():
        e0 = (NW * MAIN + wid) * CE
        pltpu.sync_copy(ei_hbm.at[0, pl.ds(e0, CE)], srcb0)
        pltpu.sync_copy(ei_hbm.at[1, pl.ds(e0, CE)], dstb0)
        pltpu.sync_copy(w_hbm.at[pl.ds(e0, CE)], wb0)
        compute(0)
        scatter(0).wait()

    plsc.subcore_barrier()
    pltpu.sync_copy(acc.at[pl.ds(sid * SLICE, SLICE)], zbuf)

    @pl.when(cid == 0)
    def _w0():
        pltpu.sync_copy(zbuf, out0_hbm.at[pl.ds(sid * SLICE, SLICE)])

    @pl.when(cid == 1)
    def _w1():
        pltpu.sync_copy(zbuf, out1_hbm.at[pl.ds(sid * SLICE, SLICE)])


def _tanh_body(v_ref, o_ref):
    o_ref[...] = jnp.tanh(v_ref[...])


def _finish_body(v_ref, p0_ref, p1_ref, o_ref):
    o_ref[...] = v_ref[...] - p0_ref[...] - p1_ref[...]


def kernel(values, weights, edge_index):
    v1 = jnp.pad(values, (0, NP - N))
    t1 = pl.pallas_call(
        _tanh_body,
        out_shape=jax.ShapeDtypeStruct((NP,), jnp.float32),
    )(v1)
    preds0, preds1 = _sc_scatter(t1, edge_index, weights)
    e1 = pl.pallas_call(
        _finish_body,
        out_shape=jax.ShapeDtypeStruct((NP,), jnp.float32),
    )(v1, preds0, preds1)
    return e1[:N]
